# baseline (speedup 1.0000x reference)
import jax
import jax.numpy as jnp
from jax import lax
from jax.experimental import pallas as pl
from jax.experimental.pallas import tpu as pltpu
from jax.experimental.pallas import tpu_sc as plsc

ROWS = 16
COLS = 20000
NC = 2
NS = 16
L = 16
SHARP = 10.0
NEG = float("-inf")

W = 640
COLS_PAD = 157 * 128
LAST_TILE = 156 * 128
C0_MAX = COLS_PAD - W
XW = 128 + W + 128
NV = W // L


def _body(x_hbm, lg_hbm, sp_hbm, pv_hbm,
          xb2, xb1, spb, pvb, lgb, sem):
    cid = lax.axis_index("c")
    sid = lax.axis_index("s")
    wid = sid * NC + cid

    pltpu.sync_copy(lg_hbm, lgb)
    logit = lgb[...]
    thresh = 1.0 / (1.0 + jnp.exp(-logit))
    t10 = SHARP * thresh
    neg = jnp.full((L,), NEG, jnp.float32)

    c0 = pl.multiple_of(jnp.minimum(wid * W, C0_MAX), 128)
    cl = pl.multiple_of(jnp.maximum(c0 - 128, 0), 128)
    cr = pl.multiple_of(jnp.minimum(c0 + W, LAST_TILE), 128)

    cps = []
    for g in range(2):
        r0 = 8 * g
        cps.append(pltpu.async_copy(
            x_hbm.at[pl.ds(r0, 8), pl.ds(cl, 128)],
            xb2.at[pl.ds(r0, 8), pl.ds(0, 128)], sem))
        cps.append(pltpu.async_copy(
            x_hbm.at[pl.ds(r0, 8), pl.ds(c0, W)],
            xb2.at[pl.ds(r0, 8), pl.ds(128, W)], sem))
        cps.append(pltpu.async_copy(
            x_hbm.at[pl.ds(r0, 8), pl.ds(cr, 128)],
            xb2.at[pl.ds(r0, 8), pl.ds(128 + W, 128)], sem))

    NM = XW // L
    ocps = []
    for g in range(2):
        r0 = 8 * g
        for cp in cps[3 * g:3 * g + 3]:
            cp.wait()

        @plsc.parallel_loop(0, 8 * NM, step=1, unroll=4)
        def mv(v):
            r = v // NM
            m = v - r * NM
            xb1[pl.ds((r0 + r) * XW + m * L, L)] = xb2[r0 + r, pl.ds(m * L, L)]

        @pl.when(wid == 0)
        def _padleft():
            for r in range(8):
                xb1[pl.ds((r0 + r) * XW + 112, L)] = neg

        @pl.when(wid == 31)
        def _padright():
            for r in range(8):
                xb1[pl.ds((r0 + r) * XW + 128 + (COLS - C0_MAX), L)] = neg

        @plsc.parallel_loop(0, 8 * NV, step=1, unroll=2)
        def step(v):
            r = v // NV
            k = v - r * NV
            base = (r0 + r) * XW + 128 + k * L
            x = xb1[pl.ds(base, L)]
            a = jnp.maximum(xb1[pl.ds(base - 2, L)], xb1[pl.ds(base - 1, L)])
            b = jnp.maximum(xb1[pl.ds(base + 1, L)], xb1[pl.ds(base + 2, L)])
            pooled = jnp.maximum(x, jnp.maximum(a, b))
            x10 = SHARP * x
            ea = jnp.exp(t10 - x10)
            eb = jnp.exp(SHARP * pooled - x10)
            sp = x / ((1.0 + ea) * (1.0 + eb))
            o = k * L
            spb[r0 + r, pl.ds(o, L)] = sp
            pvb[r0 + r, pl.ds(o, L)] = jnp.where(sp >= thresh, x, 0.0)

        for (buf, hbm) in ((spb, sp_hbm), (pvb, pv_hbm)):
            ocps.append(pltpu.async_copy(
                buf.at[pl.ds(r0, 8), pl.ds(0, W)],
                hbm.at[pl.ds(r0, 8), pl.ds(c0, W)], sem))
    for cp in ocps:
        cp.wait()


@jax.jit
def _run(peak_map, logit_vec):
    mesh = plsc.VectorSubcoreMesh(
        core_axis_name="c", subcore_axis_name="s", num_cores=NC, num_subcores=NS
    )
    f = pl.kernel(
        _body,
        out_type=(
            jax.ShapeDtypeStruct((ROWS, COLS), jnp.float32),
            jax.ShapeDtypeStruct((ROWS, COLS), jnp.float32),
        ),
        mesh=mesh,
        scratch_types=[
            pltpu.VMEM((ROWS, XW), jnp.float32),
            pltpu.VMEM((ROWS * XW,), jnp.float32),
            pltpu.VMEM((ROWS, W), jnp.float32),
            pltpu.VMEM((ROWS, W), jnp.float32),
            pltpu.VMEM((L,), jnp.float32),
            pltpu.SemaphoreType.DMA,
        ],
    )
    return f(peak_map, logit_vec)


def kernel(peak_map, logit_thresh):
    logit_vec = jnp.broadcast_to(logit_thresh.astype(jnp.float32), (L,))
    sp, pv = _run(peak_map, logit_vec)
    return sp, sp >= jax.nn.sigmoid(logit_thresh), pv


# speedup vs baseline: 1.0216x; 1.0216x over previous
import jax
import jax.numpy as jnp
from jax import lax
from jax.experimental import pallas as pl
from jax.experimental.pallas import tpu as pltpu
from jax.experimental.pallas import tpu_sc as plsc

ROWS = 16
COLS = 20000
NC = 2
NS = 16
L = 16
SHARP = 10.0
NEG = float("-inf")

W = 640
COLS_PAD = 157 * 128
LAST_TILE = 156 * 128
C0_MAX = COLS_PAD - W
XW = 128 + W + 128
NV = W // L


def _body(x_hbm, lg_hbm, sp_hbm, pv_hbm,
          xb2, xb1, spb, pvb, lgb, sem):
    cid = lax.axis_index("c")
    sid = lax.axis_index("s")
    wid = sid * NC + cid

    pltpu.sync_copy(lg_hbm, lgb)
    logit = lgb[...]
    thresh = 1.0 / (1.0 + jnp.exp(-logit))
    t10 = SHARP * thresh
    neg = jnp.full((L,), NEG, jnp.float32)

    c0 = pl.multiple_of(jnp.minimum(wid * W, C0_MAX), 128)
    cl = pl.multiple_of(jnp.maximum(c0 - 128, 0), 128)
    cr = pl.multiple_of(jnp.minimum(c0 + W, LAST_TILE), 128)

    cps = []
    for g in range(2):
        r0 = 8 * g
        cps.append(pltpu.async_copy(
            x_hbm.at[pl.ds(r0, 8), pl.ds(cl, 128)],
            xb2.at[pl.ds(r0, 8), pl.ds(0, 128)], sem))
        cps.append(pltpu.async_copy(
            x_hbm.at[pl.ds(r0, 8), pl.ds(c0, W)],
            xb2.at[pl.ds(r0, 8), pl.ds(128, W)], sem))
        cps.append(pltpu.async_copy(
            x_hbm.at[pl.ds(r0, 8), pl.ds(cr, 128)],
            xb2.at[pl.ds(r0, 8), pl.ds(128 + W, 128)], sem))

    NM = XW // L
    ocps = []
    for g in range(2):
        r0 = 8 * g
        for cp in cps[3 * g:3 * g + 3]:
            cp.wait()

        @plsc.parallel_loop(0, 8 * NM, step=1, unroll=8)
        def mv(v):
            r = v // NM
            m = v - r * NM
            xb1[pl.ds((r0 + r) * XW + m * L, L)] = xb2[r0 + r, pl.ds(m * L, L)]

        @pl.when(wid == 0)
        def _padleft():
            for r in range(8):
                xb1[pl.ds((r0 + r) * XW + 112, L)] = neg

        @pl.when(wid == 31)
        def _padright():
            for r in range(8):
                xb1[pl.ds((r0 + r) * XW + 128 + (COLS - C0_MAX), L)] = neg

        @plsc.parallel_loop(0, 8 * NV, step=1, unroll=4)
        def step(v):
            r = v // NV
            k = v - r * NV
            base = (r0 + r) * XW + 128 + k * L
            x = xb1[pl.ds(base, L)]
            a = jnp.maximum(xb1[pl.ds(base - 2, L)], xb1[pl.ds(base - 1, L)])
            b = jnp.maximum(xb1[pl.ds(base + 1, L)], xb1[pl.ds(base + 2, L)])
            pooled = jnp.maximum(x, jnp.maximum(a, b))
            x10 = SHARP * x
            ea = jnp.exp(t10 - x10)
            eb = jnp.exp(SHARP * pooled - x10)
            sp = x / ((1.0 + ea) * (1.0 + eb))
            o = k * L
            spb[r0 + r, pl.ds(o, L)] = sp
            pvb[r0 + r, pl.ds(o, L)] = jnp.where(sp >= thresh, x, 0.0)

        for (buf, hbm) in ((spb, sp_hbm), (pvb, pv_hbm)):
            ocps.append(pltpu.async_copy(
                buf.at[pl.ds(r0, 8), pl.ds(0, W)],
                hbm.at[pl.ds(r0, 8), pl.ds(c0, W)], sem))
    for cp in ocps:
        cp.wait()


@jax.jit
def _run(peak_map, logit_vec):
    mesh = plsc.VectorSubcoreMesh(
        core_axis_name="c", subcore_axis_name="s", num_cores=NC, num_subcores=NS
    )
    f = pl.kernel(
        _body,
        out_type=(
            jax.ShapeDtypeStruct((ROWS, COLS), jnp.float32),
            jax.ShapeDtypeStruct((ROWS, COLS), jnp.float32),
        ),
        mesh=mesh,
        scratch_types=[
            pltpu.VMEM((ROWS, XW), jnp.float32),
            pltpu.VMEM((ROWS * XW,), jnp.float32),
            pltpu.VMEM((ROWS, W), jnp.float32),
            pltpu.VMEM((ROWS, W), jnp.float32),
            pltpu.VMEM((L,), jnp.float32),
            pltpu.SemaphoreType.DMA,
        ],
    )
    return f(peak_map, logit_vec)


def kernel(peak_map, logit_thresh):
    logit_vec = jnp.broadcast_to(logit_thresh.astype(jnp.float32), (L,))
    sp, pv = _run(peak_map, logit_vec)
    return sp, sp >= jax.nn.sigmoid(logit_thresh), pv
